# baseline (speedup 1.0000x reference)
import functools
import math

import jax
import jax.numpy as jnp
from jax import lax
from jax.experimental import pallas as pl
from jax.experimental.pallas import tpu as pltpu
from jax.experimental.pallas import tpu_sc as plsc

SR = 16000
MAXD = 800.0
HALO = 896
B = 16
L = 64000
W = 4096
NVEC = W // 16
K = 32
NBLK = NVEC // K
NROW = 8

S1 = 0.9999999765137555
S3 = -0.16666647593489578
S5 = 0.008332899222833035
S7 = -0.00019800865307231935
S9 = 2.5904300308081957e-06
TWO_PI = float(2.0 * math.pi)
INV_2PI = float(1.0 / (2.0 * math.pi))

_MESH = plsc.VectorSubcoreMesh(core_axis_name="c", subcore_axis_name="s")


@functools.partial(
    pl.kernel,
    out_type=jax.ShapeDtypeStruct((B, L), jnp.float32),
    mesh=_MESH,
    compiler_params=pltpu.CompilerParams(needs_layout_passes=False),
    scratch_types=[
        pltpu.VMEM((NROW, HALO + W), jnp.float32),
        pltpu.VMEM((NROW, W), jnp.float32),
        pltpu.VMEM((HALO + W,), jnp.float32),
        pltpu.VMEM((5 * B,), jnp.float32),
        pltpu.VMEM((NBLK * 16,), jnp.float32),
        pltpu.VMEM((NBLK * 16,), jnp.float32),
    ],
)
def _chorus_sc(audio, params, out, buf, outbuf, rowbuf, pbuf, anchs, anchc):
    kcol = lax.axis_index("s")
    r0 = pl.multiple_of(lax.axis_index("c") * NROW, NROW)
    cs_out = jnp.minimum(kcol * W, L - W)
    shift = HALO * (kcol == 0).astype(jnp.int32)
    cs_src = pl.multiple_of(cs_out - HALO + shift, 128)
    pltpu.sync_copy(audio.at[pl.ds(r0, NROW), pl.ds(cs_src, HALO + W)], buf)
    pltpu.sync_copy(params, pbuf)
    loc0 = (HALO - shift) - cs_out
    dry0 = HALO - shift

    iota = lax.iota(jnp.int32, 16)
    iota_f = iota.astype(jnp.float32)
    cs_out_f = cs_out.astype(jnp.float32)

    def sin_reduced(u):
        f = u - u.astype(jnp.int32).astype(jnp.float32)
        k = (2.0 * f + 0.5).astype(jnp.int32)
        s = f - 0.5 * k.astype(jnp.float32)
        th = TWO_PI * s
        t2 = th * th
        p = ((((S9 * t2 + S7) * t2 + S5) * t2 + S3) * t2 + S1) * th
        return jnp.where((k & 1) == 1, -p, p)

    def row_body(r, _):
        row = r0 + r
        bvec = jnp.full((16,), row, dtype=jnp.int32)

        @plsc.parallel_loop(0, (HALO + W) // 16, step=1, unroll=4)
        def copy_row(q):
            rowbuf[pl.ds(q * 16, 16)] = buf[r, pl.ds(q * 16, 16)]

        def prow(q):
            return plsc.load_gather(
                pbuf, [jnp.full((16,), q * B, dtype=jnp.int32) + bvec])

        w2 = prow(0)
        dep = prow(1)
        centre = prow(2)
        mx = prow(3)
        omx = prow(4)
        dc = dep * centre

        w216 = w2 * (16.0 / SR)
        theta0 = w2 * ((cs_out_f + iota_f) / float(SR))
        u0 = theta0 * INV_2PI
        s0_init = sin_reduced(u0)
        c0_init = sin_reduced(u0 + 0.25)
        xB = float(K) * w216
        xB2 = xB * xB
        sB = (((-1.0 / 5040.0) * xB2 + (1.0 / 120.0)) * xB2 + (-1.0 / 6.0)) \
            * xB2 * xB + xB
        cB = (((-1.0 / 720.0) * xB2 + (1.0 / 24.0)) * xB2 + (-0.5)) * xB2 + 1.0

        def fill_anchor(m, carry):
            s0, c0 = carry
            anchs[pl.ds(m * 16, 16)] = s0
            anchc[pl.ds(m * 16, 16)] = c0
            s0n = s0 * cB + c0 * sB
            c0n = c0 * cB - s0 * sB
            return (s0n, c0n)

        lax.fori_loop(0, NBLK, fill_anchor, (s0_init, c0_init))

        @plsc.parallel_loop(0, NVEC, step=1, unroll=4)
        def body(jj):
            m16 = lax.shift_right_logical(jj, 1) & -16
            s0 = anchs[pl.ds(m16, 16)]
            c0 = anchc[pl.ds(m16, 16)]
            jk = jj & (K - 1)
            jf = jk.astype(jnp.float32)
            x = w216 * jf
            x2 = x * x
            sinx = x * (1.0 - x2 * (1.0 / 6.0))
            cosx = 1.0 - x2 * 0.5
            lfo = s0 * cosx + c0 * sinx

            delay = centre + lfo * dc
            delay = jnp.minimum(jnp.maximum(delay, 1.0), MAXD)
            li16 = jj * 16
            i_f = (cs_out_f + li16.astype(jnp.float32)) + iota_f
            rp = i_f - delay
            valid = rp >= 0.0
            rp_c = jnp.maximum(rp, 0.0)
            idx_g = rp_c.astype(jnp.int32)
            frac = rp_c - idx_g.astype(jnp.float32)
            lidx = idx_g + loc0
            lo = plsc.load_gather(rowbuf, [lidx])
            hi = plsc.load_gather(rowbuf, [lidx + 1])
            interp = lo * (1.0 - frac) + hi * frac
            delayed = jnp.where(valid, interp, 0.0)
            a = rowbuf[pl.ds(dry0 + li16, 16)]
            outbuf[r, pl.ds(li16, 16)] = a * omx + delayed * mx

        return 0

    lax.fori_loop(0, NROW, row_body, 0)
    pltpu.sync_copy(outbuf, out.at[pl.ds(r0, NROW), pl.ds(cs_out, W)])


def kernel(audio, rate_hz, depth, centre_delay_ms, feedback, mix):
    del feedback
    centre_s = centre_delay_ms.astype(jnp.float32) * (SR / 1000.0)
    params = jnp.stack([
        TWO_PI * rate_hz.astype(jnp.float32),
        depth.astype(jnp.float32),
        centre_s,
        mix.astype(jnp.float32),
        1.0 - mix.astype(jnp.float32),
    ]).reshape(-1)
    return _chorus_sc(audio.astype(jnp.float32), params)


# speedup vs baseline: 1.0892x; 1.0892x over previous
import functools
import math

import jax
import jax.numpy as jnp
from jax import lax
from jax.experimental import pallas as pl
from jax.experimental.pallas import tpu as pltpu
from jax.experimental.pallas import tpu_sc as plsc

SR = 16000
MAXD = 800.0
HALO = 896
B = 16
L = 64000
W = 4096
NVEC = W // 16
K = 32
NBLK = NVEC // K
NROW = 8

S1 = 0.9999999765137555
S3 = -0.16666647593489578
S5 = 0.008332899222833035
S7 = -0.00019800865307231935
S9 = 2.5904300308081957e-06
TWO_PI = float(2.0 * math.pi)
INV_2PI = float(1.0 / (2.0 * math.pi))

_MESH = plsc.VectorSubcoreMesh(core_axis_name="c", subcore_axis_name="s")


@functools.partial(
    pl.kernel,
    out_type=jax.ShapeDtypeStruct((B, L), jnp.float32),
    mesh=_MESH,
    compiler_params=pltpu.CompilerParams(needs_layout_passes=False),
    scratch_types=[
        pltpu.VMEM((NROW, HALO + W), jnp.float32),
        pltpu.VMEM((NROW, W), jnp.float32),
        pltpu.VMEM((HALO + W,), jnp.float32),
        pltpu.VMEM((5 * B,), jnp.float32),
        pltpu.VMEM((NROW * NBLK * 16,), jnp.float32),
        pltpu.VMEM((NROW * NBLK * 16,), jnp.float32),
        pltpu.SemaphoreType.DMA,
    ],
)
def _chorus_sc(audio, params, out, buf, outbuf, rowbuf, pbuf, anchs, anchc,
               sem):
    kcol = lax.axis_index("s")
    r0 = pl.multiple_of(lax.axis_index("c") * NROW, NROW)
    cs_out = jnp.minimum(kcol * W, L - W)
    shift = HALO * (kcol == 0).astype(jnp.int32)
    cs_src = pl.multiple_of(cs_out - HALO + shift, 128)
    in_cp = pltpu.async_copy(
        audio.at[pl.ds(r0, NROW), pl.ds(cs_src, HALO + W)], buf, sem)
    pltpu.sync_copy(params, pbuf)
    loc0 = (HALO - shift) - cs_out
    dry0 = HALO - shift

    iota = lax.iota(jnp.int32, 16)
    iota_f = iota.astype(jnp.float32)
    cs_out_f = cs_out.astype(jnp.float32)

    def sin_reduced(u):
        f = u - u.astype(jnp.int32).astype(jnp.float32)
        k = (2.0 * f + 0.5).astype(jnp.int32)
        s = f - 0.5 * k.astype(jnp.float32)
        th = TWO_PI * s
        t2 = th * th
        p = ((((S9 * t2 + S7) * t2 + S5) * t2 + S3) * t2 + S1) * th
        return jnp.where((k & 1) == 1, -p, p)

    def prow_of(row, q):
        return plsc.load_gather(
            pbuf, [jnp.full((16,), q * B, dtype=jnp.int32)
                   + jnp.full((16,), row, dtype=jnp.int32)])

    def anchor_row(r, _):
        row = r0 + r
        w2 = prow_of(row, 0)
        w216 = w2 * (16.0 / SR)
        theta0 = w2 * ((cs_out_f + iota_f) / float(SR))
        u0 = theta0 * INV_2PI
        s0_init = sin_reduced(u0)
        c0_init = sin_reduced(u0 + 0.25)
        xB = float(K) * w216
        xB2 = xB * xB
        sB = (((-1.0 / 5040.0) * xB2 + (1.0 / 120.0)) * xB2 + (-1.0 / 6.0)) \
            * xB2 * xB + xB
        cB = (((-1.0 / 720.0) * xB2 + (1.0 / 24.0)) * xB2 + (-0.5)) * xB2 + 1.0
        abase = r * (NBLK * 16)

        def fill_anchor(m, carry):
            s0, c0 = carry
            anchs[pl.ds(abase + m * 16, 16)] = s0
            anchc[pl.ds(abase + m * 16, 16)] = c0
            s0n = s0 * cB + c0 * sB
            c0n = c0 * cB - s0 * sB
            return (s0n, c0n)

        lax.fori_loop(0, NBLK, fill_anchor, (s0_init, c0_init))
        return 0

    lax.fori_loop(0, NROW, anchor_row, 0)
    in_cp.wait()

    def row_body(r, _):
        row = r0 + r

        @plsc.parallel_loop(0, (HALO + W) // 16, step=1, unroll=8)
        def copy_row(q):
            rowbuf[pl.ds(q * 16, 16)] = buf[r, pl.ds(q * 16, 16)]

        w2 = prow_of(row, 0)
        dep = prow_of(row, 1)
        centre = prow_of(row, 2)
        mx = prow_of(row, 3)
        omx = prow_of(row, 4)
        dc = dep * centre
        w216 = w2 * (16.0 / SR)
        abase = r * (NBLK * 16)

        @plsc.parallel_loop(0, NVEC, step=1, unroll=8)
        def body(jj):
            m16 = abase + (lax.shift_right_logical(jj, 1) & -16)
            s0 = anchs[pl.ds(m16, 16)]
            c0 = anchc[pl.ds(m16, 16)]
            jk = jj & (K - 1)
            jf = jk.astype(jnp.float32)
            x = w216 * jf
            x2 = x * x
            sinx = x * (1.0 - x2 * (1.0 / 6.0))
            cosx = 1.0 - x2 * 0.5
            lfo = s0 * cosx + c0 * sinx

            delay = centre + lfo * dc
            delay = jnp.minimum(jnp.maximum(delay, 1.0), MAXD)
            li16 = jj * 16
            i_f = (cs_out_f + li16.astype(jnp.float32)) + iota_f
            rp = i_f - delay
            valid = rp >= 0.0
            rp_c = jnp.maximum(rp, 0.0)
            idx_g = rp_c.astype(jnp.int32)
            frac = rp_c - idx_g.astype(jnp.float32)
            lidx = idx_g + loc0
            lo = plsc.load_gather(rowbuf, [lidx], mask=valid)
            hi = plsc.load_gather(rowbuf, [lidx + 1], mask=valid)
            delayed = lo + frac * (hi - lo)
            a = rowbuf[pl.ds(dry0 + li16, 16)]
            outbuf[r, pl.ds(li16, 16)] = a * omx + delayed * mx

        return 0

    lax.fori_loop(0, NROW, row_body, 0)
    pltpu.sync_copy(outbuf, out.at[pl.ds(r0, NROW), pl.ds(cs_out, W)])


def kernel(audio, rate_hz, depth, centre_delay_ms, feedback, mix):
    del feedback
    centre_s = centre_delay_ms.astype(jnp.float32) * (SR / 1000.0)
    params = jnp.stack([
        TWO_PI * rate_hz.astype(jnp.float32),
        depth.astype(jnp.float32),
        centre_s,
        mix.astype(jnp.float32),
        1.0 - mix.astype(jnp.float32),
    ]).reshape(-1)
    return _chorus_sc(audio.astype(jnp.float32), params)
